# Initial kernel scaffold; baseline (speedup 1.0000x reference)
import jax
import jax.numpy as jnp
from jax.experimental import pallas as pl


def kernel(x, edge_index, batch, W1, b1, W2, b2, W3, b3, W4, b4, lin_w, lin_b):
    raise NotImplementedError("write your pallas kernel here")


# speedup vs baseline: 15.1928x; 15.1928x over previous
import functools

import jax
import jax.numpy as jnp
from jax import lax
from jax.experimental import pallas as pl
from jax.experimental.pallas import tpu as pltpu
from jax.experimental.pallas import tpu_sc as plsc

N = 10000
D = 128
HD = D // 2
E = 320000
G = 64
NC = 2
NS = 16
NW = NC * NS
CH = 125
NCH32 = E // NW // CH
NCH16 = E // NS // CH
NPAD = 10240
PT = NPAD // NS

BM = 2000
NBLK = N // BM

_mesh = plsc.VectorSubcoreMesh(
    core_axis_name="c", subcore_axis_name="s", num_cores=NC, num_subcores=NS)

_f32 = jnp.float32


@functools.partial(
    pl.kernel,
    out_type=[jax.ShapeDtypeStruct((NPAD,), _f32),
              jax.ShapeDtypeStruct((NPAD,), _f32)],
    mesh=_mesh,
    scratch_types=[
        pltpu.VMEM((NCH32, CH), jnp.int32),
        pltpu.VMEM((PT,), _f32),
        pltpu.VMEM((128,), _f32),
        pltpu.VMEM_SHARED((NPAD,), _f32),
    ],
)
def _deg_kernel(dst_hbm, out0_hbm, out1_hbm, dst_v, zbuf_v, ones_v, acc_s):
    c = lax.axis_index("c")
    s = lax.axis_index("s")
    wid = c * NS + s
    pltpu.sync_copy(dst_hbm.at[wid], dst_v)

    def zfill(i, carry):
        zbuf_v[pl.ds(i * 16, 16)] = jnp.zeros((16,), _f32)
        return carry
    lax.fori_loop(0, PT // 16, zfill, 0)

    def ofill(i, carry):
        ones_v[pl.ds(i * 16, 16)] = jnp.ones((16,), _f32)
        return carry
    lax.fori_loop(0, 8, ofill, 0)

    pltpu.sync_copy(zbuf_v, acc_s.at[pl.ds(s * PT, PT)])
    plsc.subcore_barrier()

    def body(j, carry):
        pltpu.sync_copy(ones_v.at[pl.ds(0, CH)], acc_s.at[dst_v.at[j]],
                        add=True)
        return carry
    lax.fori_loop(0, NCH32, body, 0)

    plsc.subcore_barrier()

    @pl.when(c == 0)
    def _():
        pltpu.sync_copy(acc_s.at[pl.ds(s * PT, PT)],
                        out0_hbm.at[pl.ds(s * PT, PT)])

    @pl.when(c == 1)
    def _():
        pltpu.sync_copy(acc_s.at[pl.ds(s * PT, PT)],
                        out1_hbm.at[pl.ds(s * PT, PT)])


@functools.partial(
    pl.kernel,
    out_type=[jax.ShapeDtypeStruct((NPAD, HD), _f32),
              jax.ShapeDtypeStruct((NPAD, HD), _f32)],
    mesh=_mesh,
    scratch_types=[
        pltpu.VMEM((NCH16, CH), jnp.int32),
        pltpu.VMEM((NCH16, CH), jnp.int32),
        pltpu.VMEM((CH, HD), _f32),
        pltpu.VMEM((128, HD), _f32),
        pltpu.VMEM_SHARED((NPAD, HD), _f32),
        pltpu.SemaphoreType.DMA,
    ],
    compiler_params=pltpu.CompilerParams(use_tc_tiling_on_sc=False),
)
def _edge_kernel(yl_hbm, yr_hbm, src_hbm, dst_hbm, out0_hbm, out1_hbm,
                 src_v, dst_v, rows_v, zrows_v, z_s, sem):
    c = lax.axis_index("c")
    s = lax.axis_index("s")
    pltpu.sync_copy(src_hbm.at[s], src_v)
    pltpu.sync_copy(dst_hbm.at[s], dst_v)

    def zfill(i, carry):
        zrows_v[i // 4, pl.ds((i % 4) * 16, 16)] = jnp.zeros((16,), _f32)
        return carry
    lax.fori_loop(0, 128 * 4, zfill, 0)

    def zcopy(k, carry):
        pltpu.sync_copy(zrows_v, z_s.at[pl.ds(s * PT + k * 128, 128)])
        return carry
    lax.fori_loop(0, PT // 128, zcopy, 0)
    plsc.subcore_barrier()

    @pl.when(c == 0)
    def _():
        def body(j, carry):
            pltpu.async_copy(yl_hbm.at[src_v.at[j]], rows_v, sem).wait()
            pltpu.sync_copy(rows_v, z_s.at[dst_v.at[j]], add=True)
            return carry
        lax.fori_loop(0, NCH16, body, 0)

    @pl.when(c == 1)
    def _():
        def body(j, carry):
            pltpu.async_copy(yr_hbm.at[src_v.at[j]], rows_v, sem).wait()
            pltpu.sync_copy(rows_v, z_s.at[dst_v.at[j]], add=True)
            return carry
        lax.fori_loop(0, NCH16, body, 0)

    plsc.subcore_barrier()

    @pl.when(c == 0)
    def _():
        pltpu.sync_copy(z_s.at[pl.ds(s * PT, PT)],
                        out0_hbm.at[pl.ds(s * PT, PT)])

    @pl.when(c == 1)
    def _():
        pltpu.sync_copy(z_s.at[pl.ds(s * PT, PT)],
                        out1_hbm.at[pl.ds(s * PT, PT)])


@functools.partial(
    pl.kernel,
    out_type=[jax.ShapeDtypeStruct((NPAD,), _f32),
              jax.ShapeDtypeStruct((NPAD,), _f32)],
    mesh=_mesh,
    scratch_types=[
        pltpu.VMEM((NCH32, CH), jnp.int32),
        pltpu.VMEM((NCH32, CH), jnp.int32),
        pltpu.VMEM((CH,), _f32),
        pltpu.VMEM((PT,), _f32),
        pltpu.VMEM_SHARED((NPAD,), _f32),
        pltpu.SemaphoreType.DMA,
    ],
)
def _scalar_edge_kernel(t_hbm, src_hbm, dst_hbm, out0_hbm, out1_hbm,
                        src_v, dst_v, vals_v, zbuf_v, acc_s, sem):
    c = lax.axis_index("c")
    s = lax.axis_index("s")
    wid = c * NS + s
    pltpu.sync_copy(src_hbm.at[wid], src_v)
    pltpu.sync_copy(dst_hbm.at[wid], dst_v)

    def zfill(i, carry):
        zbuf_v[pl.ds(i * 16, 16)] = jnp.zeros((16,), _f32)
        return carry
    lax.fori_loop(0, PT // 16, zfill, 0)

    pltpu.sync_copy(zbuf_v, acc_s.at[pl.ds(s * PT, PT)])
    plsc.subcore_barrier()

    def body(j, carry):
        pltpu.async_copy(t_hbm.at[src_v.at[j]], vals_v, sem).wait()
        pltpu.sync_copy(vals_v, acc_s.at[dst_v.at[j]], add=True)
        return carry
    lax.fori_loop(0, NCH32, body, 0)

    plsc.subcore_barrier()

    @pl.when(c == 0)
    def _():
        pltpu.sync_copy(acc_s.at[pl.ds(s * PT, PT)],
                        out0_hbm.at[pl.ds(s * PT, PT)])

    @pl.when(c == 1)
    def _():
        pltpu.sync_copy(acc_s.at[pl.ds(s * PT, PT)],
                        out1_hbm.at[pl.ds(s * PT, PT)])


def _mm_scale_body(x_ref, w_ref, d0_ref, d1_ref, y_ref, dinv_ref):
    deg = d0_ref[...] + d1_ref[...] + 1.0
    dinv = lax.rsqrt(deg)
    dinv_ref[...] = dinv
    y_ref[...] = dinv * jnp.dot(x_ref[...], w_ref[...],
                                preferred_element_type=_f32)


def _first_layer(x, W1, d0, d1):
    return pl.pallas_call(
        _mm_scale_body,
        grid=(NBLK,),
        in_specs=[
            pl.BlockSpec((BM, D), lambda i: (i, 0)),
            pl.BlockSpec((D, D), lambda i: (0, 0)),
            pl.BlockSpec((BM, 1), lambda i: (i, 0)),
            pl.BlockSpec((BM, 1), lambda i: (i, 0)),
        ],
        out_specs=[
            pl.BlockSpec((BM, D), lambda i: (i, 0)),
            pl.BlockSpec((BM, 1), lambda i: (i, 0)),
        ],
        out_shape=[
            jax.ShapeDtypeStruct((N, D), _f32),
            jax.ShapeDtypeStruct((N, 1), _f32),
        ],
    )(x, W1, d0, d1)


def _layer_body(z0_ref, z1_ref, y_ref, dinv_ref, b_ref, w_ref, out_ref):
    dinv = dinv_ref[...]
    z = jnp.concatenate([z0_ref[...], z1_ref[...]], axis=1)
    h = dinv * (z + y_ref[...]) + b_ref[...]
    h = jnp.maximum(h, 0.0)
    out_ref[...] = dinv * jnp.dot(h, w_ref[...], preferred_element_type=_f32)


def _mid_layer(z0, z1, y, dinv, b, Wn):
    return pl.pallas_call(
        _layer_body,
        grid=(NBLK,),
        in_specs=[
            pl.BlockSpec((BM, HD), lambda i: (i, 0)),
            pl.BlockSpec((BM, HD), lambda i: (i, 0)),
            pl.BlockSpec((BM, D), lambda i: (i, 0)),
            pl.BlockSpec((BM, 1), lambda i: (i, 0)),
            pl.BlockSpec((1, D), lambda i: (0, 0)),
            pl.BlockSpec((D, D), lambda i: (0, 0)),
        ],
        out_specs=pl.BlockSpec((BM, D), lambda i: (i, 0)),
        out_shape=jax.ShapeDtypeStruct((N, D), _f32),
    )(z0, z1, y, dinv, b, Wn)


def _final_node_body(z0_ref, z1_ref, y_ref, dinv_ref, b_ref, w4_ref, lw_ref,
                     t_ref):
    dinv = dinv_ref[...]
    z = jnp.concatenate([z0_ref[...], z1_ref[...]], axis=1)
    h = dinv * (z + y_ref[...]) + b_ref[...]
    h = jnp.maximum(h, 0.0)
    v = jnp.dot(w4_ref[...], lw_ref[...], preferred_element_type=_f32)
    t_ref[...] = dinv * jnp.dot(h, v, preferred_element_type=_f32)


def _final_node(z0, z1, y, dinv, b, W4, lin_w):
    return pl.pallas_call(
        _final_node_body,
        grid=(NBLK,),
        in_specs=[
            pl.BlockSpec((BM, HD), lambda i: (i, 0)),
            pl.BlockSpec((BM, HD), lambda i: (i, 0)),
            pl.BlockSpec((BM, D), lambda i: (i, 0)),
            pl.BlockSpec((BM, 1), lambda i: (i, 0)),
            pl.BlockSpec((1, D), lambda i: (0, 0)),
            pl.BlockSpec((D, D), lambda i: (0, 0)),
            pl.BlockSpec((D, 1), lambda i: (0, 0)),
        ],
        out_specs=pl.BlockSpec((BM, 1), lambda i: (i, 0)),
        out_shape=jax.ShapeDtypeStruct((N, 1), _f32),
    )(z0, z1, y, dinv, b, W4, lin_w)


def _pool_body(u0_ref, u1_ref, t_ref, dinv_ref, batch_ref, b4_ref, lw_ref,
               lb_ref, out_ref, acc_ref, cnt_ref):
    i = pl.program_id(0)

    @pl.when(i == 0)
    def _():
        acc_ref[...] = jnp.zeros_like(acc_ref)
        cnt_ref[...] = jnp.zeros_like(cnt_ref)

    u = dinv_ref[...] * (u0_ref[...] + u1_ref[...] + t_ref[...])
    gids = lax.broadcasted_iota(jnp.int32, (BM, G), 1)
    mask = (batch_ref[...] == gids).astype(_f32)
    acc_ref[...] += jnp.sum(mask * u, axis=0, keepdims=True)
    cnt_ref[...] += jnp.sum(mask, axis=0, keepdims=True)

    @pl.when(i == pl.num_programs(0) - 1)
    def _():
        c = jnp.sum(b4_ref[...] * lw_ref[...]) + lb_ref[0, 0]
        res = acc_ref[...] / jnp.maximum(cnt_ref[...], 1.0) + c
        out_ref[...] = jnp.broadcast_to(res, (8, G))


def _pool(u0, u1, t, dinv, batch2d, b4r, lin_w, lb):
    return pl.pallas_call(
        _pool_body,
        grid=(NBLK,),
        in_specs=[
            pl.BlockSpec((BM, 1), lambda i: (i, 0)),
            pl.BlockSpec((BM, 1), lambda i: (i, 0)),
            pl.BlockSpec((BM, 1), lambda i: (i, 0)),
            pl.BlockSpec((BM, 1), lambda i: (i, 0)),
            pl.BlockSpec((BM, 1), lambda i: (i, 0)),
            pl.BlockSpec((D, 1), lambda i: (0, 0)),
            pl.BlockSpec((D, 1), lambda i: (0, 0)),
            pl.BlockSpec((1, 1), lambda i: (0, 0), memory_space=pltpu.SMEM),
        ],
        out_specs=pl.BlockSpec((8, G), lambda i: (0, 0)),
        out_shape=jax.ShapeDtypeStruct((8, G), _f32),
        scratch_shapes=[
            pltpu.VMEM((1, G), _f32),
            pltpu.VMEM((1, G), _f32),
        ],
    )(u0, u1, t, dinv, batch2d, b4r, lin_w, lb)


def kernel(x, edge_index, batch, W1, b1, W2, b2, W3, b3, W4, b4, lin_w, lin_b):
    src32 = edge_index[0].reshape(NW, NCH32, CH)
    dst32 = edge_index[1].reshape(NW, NCH32, CH)
    src16 = edge_index[0].reshape(NS, NCH16, CH)
    dst16 = edge_index[1].reshape(NS, NCH16, CH)

    deg0, deg1 = _deg_kernel(dst32)
    d0 = deg0[:N, None]
    d1 = deg1[:N, None]

    y1, dinv = _first_layer(x, W1, d0, d1)
    z0, z1 = _edge_kernel(y1[:, :HD], y1[:, HD:], src16, dst16)
    y2 = _mid_layer(z0[:N], z1[:N], y1, dinv, b1.reshape(1, D), W2)
    z0, z1 = _edge_kernel(y2[:, :HD], y2[:, HD:], src16, dst16)
    y3 = _mid_layer(z0[:N], z1[:N], y2, dinv, b2.reshape(1, D), W3)
    z0, z1 = _edge_kernel(y3[:, :HD], y3[:, HD:], src16, dst16)
    t = _final_node(z0[:N], z1[:N], y3, dinv, b3.reshape(1, D), W4, lin_w)

    u0, u1 = _scalar_edge_kernel(t.reshape(N), src32, dst32)
    out8 = _pool(u0[:N, None], u1[:N, None], t, dinv,
                 batch.reshape(N, 1), b4.reshape(D, 1), lin_w,
                 lin_b.reshape(1, 1))
    return out8[0].reshape(G, 1)


# speedup vs baseline: 18.3883x; 1.2103x over previous
import functools

import jax
import jax.numpy as jnp
from jax import lax
from jax.experimental import pallas as pl
from jax.experimental.pallas import tpu as pltpu
from jax.experimental.pallas import tpu_sc as plsc

N = 10000
D = 128
HD = D // 2
E = 320000
G = 64
NC = 2
NS = 16
NW = NC * NS
CH = 125
NCH32 = E // NW // CH
NCH16 = E // NS // CH
NPAD = 10240
PT = NPAD // NS

BM = 2000
NBLK = N // BM

_mesh = plsc.VectorSubcoreMesh(
    core_axis_name="c", subcore_axis_name="s", num_cores=NC, num_subcores=NS)

_f32 = jnp.float32


@functools.partial(
    pl.kernel,
    out_type=[jax.ShapeDtypeStruct((NPAD,), _f32),
              jax.ShapeDtypeStruct((NPAD,), _f32)],
    mesh=_mesh,
    scratch_types=[
        pltpu.VMEM((NCH32, CH), jnp.int32),
        pltpu.VMEM((PT,), _f32),
        pltpu.VMEM((128,), _f32),
        pltpu.VMEM_SHARED((NPAD,), _f32),
    ],
)
def _deg_kernel(dst_hbm, out0_hbm, out1_hbm, dst_v, zbuf_v, ones_v, acc_s):
    c = lax.axis_index("c")
    s = lax.axis_index("s")
    wid = c * NS + s
    pltpu.sync_copy(dst_hbm.at[wid], dst_v)

    def zfill(i, carry):
        zbuf_v[pl.ds(i * 16, 16)] = jnp.zeros((16,), _f32)
        return carry
    lax.fori_loop(0, PT // 16, zfill, 0)

    def ofill(i, carry):
        ones_v[pl.ds(i * 16, 16)] = jnp.ones((16,), _f32)
        return carry
    lax.fori_loop(0, 8, ofill, 0)

    pltpu.sync_copy(zbuf_v, acc_s.at[pl.ds(s * PT, PT)])
    plsc.subcore_barrier()

    def body(j, carry):
        pltpu.sync_copy(ones_v.at[pl.ds(0, CH)], acc_s.at[dst_v.at[j]],
                        add=True)
        return carry
    lax.fori_loop(0, NCH32, body, 0)

    plsc.subcore_barrier()

    @pl.when(c == 0)
    def _():
        pltpu.sync_copy(acc_s.at[pl.ds(s * PT, PT)],
                        out0_hbm.at[pl.ds(s * PT, PT)])

    @pl.when(c == 1)
    def _():
        pltpu.sync_copy(acc_s.at[pl.ds(s * PT, PT)],
                        out1_hbm.at[pl.ds(s * PT, PT)])


@functools.partial(
    pl.kernel,
    out_type=[jax.ShapeDtypeStruct((NPAD, HD), _f32),
              jax.ShapeDtypeStruct((NPAD, HD), _f32)],
    mesh=_mesh,
    scratch_types=[
        pltpu.VMEM((NCH16, CH), jnp.int32),
        pltpu.VMEM((NCH16, CH), jnp.int32),
        pltpu.VMEM((CH, HD), _f32),
        pltpu.VMEM((CH, HD), _f32),
        pltpu.VMEM((128, HD), _f32),
        pltpu.VMEM_SHARED((NPAD, HD), _f32),
        pltpu.SemaphoreType.DMA,
        pltpu.SemaphoreType.DMA,
        pltpu.SemaphoreType.DMA,
        pltpu.SemaphoreType.DMA,
    ],
    compiler_params=pltpu.CompilerParams(use_tc_tiling_on_sc=False),
)
def _edge_kernel(yl_hbm, yr_hbm, src_hbm, dst_hbm, out0_hbm, out1_hbm,
                 src_v, dst_v, rows_a, rows_b, zrows_v, z_s,
                 gsem_a, gsem_b, ssem_a, ssem_b):
    c = lax.axis_index("c")
    s = lax.axis_index("s")
    pltpu.sync_copy(src_hbm.at[s], src_v)
    pltpu.sync_copy(dst_hbm.at[s], dst_v)

    def zfill(i, carry):
        zrows_v[i // 4, pl.ds((i % 4) * 16, 16)] = jnp.zeros((16,), _f32)
        return carry
    lax.fori_loop(0, 128 * 4, zfill, 0)

    def zcopy(k, carry):
        pltpu.sync_copy(zrows_v, z_s.at[pl.ds(s * PT + k * 128, 128)])
        return carry
    lax.fori_loop(0, PT // 128, zcopy, 0)
    plsc.subcore_barrier()

    def run(y_hbm):
        pltpu.async_copy(y_hbm.at[src_v.at[0]], rows_a, gsem_a)

        def body(j2, carry):
            j = j2 * 2
            pltpu.make_async_copy(y_hbm.at[src_v.at[j]], rows_a, gsem_a).wait()

            @pl.when(j2 > 0)
            def _():
                pltpu.make_async_copy(rows_b, z_s.at[dst_v.at[j]],
                                      ssem_b).wait()
            pltpu.async_copy(y_hbm.at[src_v.at[j + 1]], rows_b, gsem_b)
            pltpu.async_copy(rows_a, z_s.at[dst_v.at[j]], ssem_a, add=True)
            pltpu.make_async_copy(y_hbm.at[src_v.at[j + 1]], rows_b,
                                  gsem_b).wait()

            @pl.when(j2 < NCH16 // 2 - 1)
            def _():
                pltpu.make_async_copy(rows_a, z_s.at[dst_v.at[j]],
                                      ssem_a).wait()
                pltpu.async_copy(y_hbm.at[src_v.at[j + 2]], rows_a, gsem_a)
            pltpu.async_copy(rows_b, z_s.at[dst_v.at[j + 1]], ssem_b,
                             add=True)
            return carry
        lax.fori_loop(0, NCH16 // 2, body, 0)
        pltpu.make_async_copy(rows_a, z_s.at[dst_v.at[0]], ssem_a).wait()
        pltpu.make_async_copy(rows_b, z_s.at[dst_v.at[0]], ssem_b).wait()

    @pl.when(c == 0)
    def _():
        run(yl_hbm)

    @pl.when(c == 1)
    def _():
        run(yr_hbm)

    plsc.subcore_barrier()

    @pl.when(c == 0)
    def _():
        pltpu.sync_copy(z_s.at[pl.ds(s * PT, PT)],
                        out0_hbm.at[pl.ds(s * PT, PT)])

    @pl.when(c == 1)
    def _():
        pltpu.sync_copy(z_s.at[pl.ds(s * PT, PT)],
                        out1_hbm.at[pl.ds(s * PT, PT)])


@functools.partial(
    pl.kernel,
    out_type=[jax.ShapeDtypeStruct((NPAD,), _f32),
              jax.ShapeDtypeStruct((NPAD,), _f32)],
    mesh=_mesh,
    scratch_types=[
        pltpu.VMEM((NCH32, CH), jnp.int32),
        pltpu.VMEM((NCH32, CH), jnp.int32),
        pltpu.VMEM((CH,), _f32),
        pltpu.VMEM((CH,), _f32),
        pltpu.VMEM((PT,), _f32),
        pltpu.VMEM_SHARED((NPAD,), _f32),
        pltpu.SemaphoreType.DMA,
        pltpu.SemaphoreType.DMA,
        pltpu.SemaphoreType.DMA,
        pltpu.SemaphoreType.DMA,
    ],
)
def _scalar_edge_kernel(t_hbm, src_hbm, dst_hbm, out0_hbm, out1_hbm,
                        src_v, dst_v, vals_a, vals_b, zbuf_v, acc_s,
                        gsem_a, gsem_b, ssem_a, ssem_b):
    c = lax.axis_index("c")
    s = lax.axis_index("s")
    wid = c * NS + s
    pltpu.sync_copy(src_hbm.at[wid], src_v)
    pltpu.sync_copy(dst_hbm.at[wid], dst_v)

    def zfill(i, carry):
        zbuf_v[pl.ds(i * 16, 16)] = jnp.zeros((16,), _f32)
        return carry
    lax.fori_loop(0, PT // 16, zfill, 0)

    pltpu.sync_copy(zbuf_v, acc_s.at[pl.ds(s * PT, PT)])
    plsc.subcore_barrier()

    pltpu.async_copy(t_hbm.at[src_v.at[0]], vals_a, gsem_a)

    def body(j2, carry):
        j = j2 * 2
        pltpu.make_async_copy(t_hbm.at[src_v.at[j]], vals_a, gsem_a).wait()

        @pl.when(j2 > 0)
        def _():
            pltpu.make_async_copy(vals_b, acc_s.at[dst_v.at[j]], ssem_b).wait()
        pltpu.async_copy(t_hbm.at[src_v.at[j + 1]], vals_b, gsem_b)
        pltpu.async_copy(vals_a, acc_s.at[dst_v.at[j]], ssem_a, add=True)

        pltpu.make_async_copy(t_hbm.at[src_v.at[j + 1]], vals_b, gsem_b).wait()

        @pl.when(j2 < NCH32 // 2 - 1)
        def _():
            pltpu.make_async_copy(vals_a, acc_s.at[dst_v.at[j]], ssem_a).wait()
            pltpu.async_copy(t_hbm.at[src_v.at[j + 2]], vals_a, gsem_a)
        pltpu.async_copy(vals_b, acc_s.at[dst_v.at[j + 1]], ssem_b, add=True)
        return carry
    lax.fori_loop(0, NCH32 // 2, body, 0)
    pltpu.make_async_copy(vals_a, acc_s.at[dst_v.at[0]], ssem_a).wait()
    pltpu.make_async_copy(vals_b, acc_s.at[dst_v.at[0]], ssem_b).wait()

    plsc.subcore_barrier()

    @pl.when(c == 0)
    def _():
        pltpu.sync_copy(acc_s.at[pl.ds(s * PT, PT)],
                        out0_hbm.at[pl.ds(s * PT, PT)])

    @pl.when(c == 1)
    def _():
        pltpu.sync_copy(acc_s.at[pl.ds(s * PT, PT)],
                        out1_hbm.at[pl.ds(s * PT, PT)])


def _mm_scale_body(x_ref, w_ref, d0_ref, d1_ref, y_ref, dinv_ref):
    deg = d0_ref[...] + d1_ref[...] + 1.0
    dinv = lax.rsqrt(deg)
    dinv_ref[...] = dinv
    y_ref[...] = dinv * jnp.dot(x_ref[...], w_ref[...],
                                preferred_element_type=_f32,
                                precision=lax.Precision.HIGHEST)


def _first_layer(x, W1, d0, d1):
    return pl.pallas_call(
        _mm_scale_body,
        grid=(NBLK,),
        in_specs=[
            pl.BlockSpec((BM, D), lambda i: (i, 0)),
            pl.BlockSpec((D, D), lambda i: (0, 0)),
            pl.BlockSpec((BM, 1), lambda i: (i, 0)),
            pl.BlockSpec((BM, 1), lambda i: (i, 0)),
        ],
        out_specs=[
            pl.BlockSpec((BM, D), lambda i: (i, 0)),
            pl.BlockSpec((BM, 1), lambda i: (i, 0)),
        ],
        out_shape=[
            jax.ShapeDtypeStruct((N, D), _f32),
            jax.ShapeDtypeStruct((N, 1), _f32),
        ],
    )(x, W1, d0, d1)


def _layer_body(z0_ref, z1_ref, y_ref, dinv_ref, b_ref, w_ref, out_ref):
    dinv = dinv_ref[...]
    z = jnp.concatenate([z0_ref[...], z1_ref[...]], axis=1)
    h = dinv * (z + y_ref[...]) + b_ref[...]
    h = jnp.maximum(h, 0.0)
    out_ref[...] = dinv * jnp.dot(h, w_ref[...], preferred_element_type=_f32,
                                precision=lax.Precision.HIGHEST)


def _mid_layer(z0, z1, y, dinv, b, Wn):
    return pl.pallas_call(
        _layer_body,
        grid=(NBLK,),
        in_specs=[
            pl.BlockSpec((BM, HD), lambda i: (i, 0)),
            pl.BlockSpec((BM, HD), lambda i: (i, 0)),
            pl.BlockSpec((BM, D), lambda i: (i, 0)),
            pl.BlockSpec((BM, 1), lambda i: (i, 0)),
            pl.BlockSpec((1, D), lambda i: (0, 0)),
            pl.BlockSpec((D, D), lambda i: (0, 0)),
        ],
        out_specs=pl.BlockSpec((BM, D), lambda i: (i, 0)),
        out_shape=jax.ShapeDtypeStruct((N, D), _f32),
    )(z0, z1, y, dinv, b, Wn)


def _final_node_body(z0_ref, z1_ref, y_ref, dinv_ref, b_ref, w4_ref, lw_ref,
                     t_ref):
    dinv = dinv_ref[...]
    z = jnp.concatenate([z0_ref[...], z1_ref[...]], axis=1)
    h = dinv * (z + y_ref[...]) + b_ref[...]
    h = jnp.maximum(h, 0.0)
    v = jnp.dot(w4_ref[...], lw_ref[...], preferred_element_type=_f32,
                                precision=lax.Precision.HIGHEST)
    t_ref[...] = dinv * jnp.dot(h, v, preferred_element_type=_f32,
                                precision=lax.Precision.HIGHEST)


def _final_node(z0, z1, y, dinv, b, W4, lin_w):
    return pl.pallas_call(
        _final_node_body,
        grid=(NBLK,),
        in_specs=[
            pl.BlockSpec((BM, HD), lambda i: (i, 0)),
            pl.BlockSpec((BM, HD), lambda i: (i, 0)),
            pl.BlockSpec((BM, D), lambda i: (i, 0)),
            pl.BlockSpec((BM, 1), lambda i: (i, 0)),
            pl.BlockSpec((1, D), lambda i: (0, 0)),
            pl.BlockSpec((D, D), lambda i: (0, 0)),
            pl.BlockSpec((D, 1), lambda i: (0, 0)),
        ],
        out_specs=pl.BlockSpec((BM, 1), lambda i: (i, 0)),
        out_shape=jax.ShapeDtypeStruct((N, 1), _f32),
    )(z0, z1, y, dinv, b, W4, lin_w)


def _pool_body(u0_ref, u1_ref, t_ref, dinv_ref, batch_ref, b4_ref, lw_ref,
               lb_ref, out_ref, acc_ref, cnt_ref):
    i = pl.program_id(0)

    @pl.when(i == 0)
    def _():
        acc_ref[...] = jnp.zeros_like(acc_ref)
        cnt_ref[...] = jnp.zeros_like(cnt_ref)

    u = dinv_ref[...] * (u0_ref[...] + u1_ref[...] + t_ref[...])
    gids = lax.broadcasted_iota(jnp.int32, (BM, G), 1)
    mask = (batch_ref[...] == gids).astype(_f32)
    acc_ref[...] += jnp.sum(mask * u, axis=0, keepdims=True)
    cnt_ref[...] += jnp.sum(mask, axis=0, keepdims=True)

    @pl.when(i == pl.num_programs(0) - 1)
    def _():
        c = jnp.sum(b4_ref[...] * lw_ref[...]) + lb_ref[0, 0]
        res = acc_ref[...] / jnp.maximum(cnt_ref[...], 1.0) + c
        out_ref[...] = jnp.broadcast_to(res, (8, G))


def _pool(u0, u1, t, dinv, batch2d, b4r, lin_w, lb):
    return pl.pallas_call(
        _pool_body,
        grid=(NBLK,),
        in_specs=[
            pl.BlockSpec((BM, 1), lambda i: (i, 0)),
            pl.BlockSpec((BM, 1), lambda i: (i, 0)),
            pl.BlockSpec((BM, 1), lambda i: (i, 0)),
            pl.BlockSpec((BM, 1), lambda i: (i, 0)),
            pl.BlockSpec((BM, 1), lambda i: (i, 0)),
            pl.BlockSpec((D, 1), lambda i: (0, 0)),
            pl.BlockSpec((D, 1), lambda i: (0, 0)),
            pl.BlockSpec((1, 1), lambda i: (0, 0), memory_space=pltpu.SMEM),
        ],
        out_specs=pl.BlockSpec((8, G), lambda i: (0, 0)),
        out_shape=jax.ShapeDtypeStruct((8, G), _f32),
        scratch_shapes=[
            pltpu.VMEM((1, G), _f32),
            pltpu.VMEM((1, G), _f32),
        ],
    )(u0, u1, t, dinv, batch2d, b4r, lin_w, lb)


def kernel(x, edge_index, batch, W1, b1, W2, b2, W3, b3, W4, b4, lin_w, lin_b):
    src32 = edge_index[0].reshape(NW, NCH32, CH)
    dst32 = edge_index[1].reshape(NW, NCH32, CH)
    src16 = edge_index[0].reshape(NS, NCH16, CH)
    dst16 = edge_index[1].reshape(NS, NCH16, CH)

    deg0, deg1 = _deg_kernel(dst32)
    d0 = deg0[:N, None]
    d1 = deg1[:N, None]

    y1, dinv = _first_layer(x, W1, d0, d1)
    z0, z1 = _edge_kernel(y1[:, :HD], y1[:, HD:], src16, dst16)
    y2 = _mid_layer(z0[:N], z1[:N], y1, dinv, b1.reshape(1, D), W2)
    z0, z1 = _edge_kernel(y2[:, :HD], y2[:, HD:], src16, dst16)
    y3 = _mid_layer(z0[:N], z1[:N], y2, dinv, b2.reshape(1, D), W3)
    z0, z1 = _edge_kernel(y3[:, :HD], y3[:, HD:], src16, dst16)
    t = _final_node(z0[:N], z1[:N], y3, dinv, b3.reshape(1, D), W4, lin_w)

    u0, u1 = _scalar_edge_kernel(t.reshape(N), src32, dst32)
    out8 = _pool(u0[:N, None], u1[:N, None], t, dinv,
                 batch.reshape(N, 1), b4.reshape(D, 1), lin_w,
                 lin_b.reshape(1, 1))
    return out8[0].reshape(G, 1)


# speedup vs baseline: 24.1731x; 1.3146x over previous
import functools

import jax
import jax.numpy as jnp
from jax import lax
from jax.experimental import pallas as pl
from jax.experimental.pallas import tpu as pltpu
from jax.experimental.pallas import tpu_sc as plsc

N = 10000
D = 128
HD = D // 2
E = 320000
G = 64
NC = 2
NS = 16
NW = NC * NS
CH = 125
NCH32 = E // NW // CH
NCH16 = E // NS // CH
NPAD = 10240
PT = NPAD // NS

BM = 2000
NBLK = N // BM

_mesh = plsc.VectorSubcoreMesh(
    core_axis_name="c", subcore_axis_name="s", num_cores=NC, num_subcores=NS)

_f32 = jnp.float32


@functools.partial(
    pl.kernel,
    out_type=[jax.ShapeDtypeStruct((NPAD,), _f32),
              jax.ShapeDtypeStruct((NPAD,), _f32)],
    mesh=_mesh,
    scratch_types=[
        pltpu.VMEM((NCH32, CH), jnp.int32),
        pltpu.VMEM((PT,), _f32),
        pltpu.VMEM((128,), _f32),
        pltpu.VMEM_SHARED((NPAD,), _f32),
    ],
)
def _deg_kernel(dst_hbm, out0_hbm, out1_hbm, dst_v, zbuf_v, ones_v, acc_s):
    c = lax.axis_index("c")
    s = lax.axis_index("s")
    wid = c * NS + s
    pltpu.sync_copy(dst_hbm.at[wid], dst_v)

    def zfill(i, carry):
        zbuf_v[pl.ds(i * 16, 16)] = jnp.zeros((16,), _f32)
        return carry
    lax.fori_loop(0, PT // 16, zfill, 0)

    def ofill(i, carry):
        ones_v[pl.ds(i * 16, 16)] = jnp.ones((16,), _f32)
        return carry
    lax.fori_loop(0, 8, ofill, 0)

    pltpu.sync_copy(zbuf_v, acc_s.at[pl.ds(s * PT, PT)])
    plsc.subcore_barrier()

    def body(j, carry):
        pltpu.sync_copy(ones_v.at[pl.ds(0, CH)], acc_s.at[dst_v.at[j]],
                        add=True)
        return carry
    lax.fori_loop(0, NCH32, body, 0)

    plsc.subcore_barrier()

    @pl.when(c == 0)
    def _():
        pltpu.sync_copy(acc_s.at[pl.ds(s * PT, PT)],
                        out0_hbm.at[pl.ds(s * PT, PT)])

    @pl.when(c == 1)
    def _():
        pltpu.sync_copy(acc_s.at[pl.ds(s * PT, PT)],
                        out1_hbm.at[pl.ds(s * PT, PT)])


@functools.partial(
    pl.kernel,
    out_type=[jax.ShapeDtypeStruct((NPAD, HD), _f32),
              jax.ShapeDtypeStruct((NPAD, HD), _f32)],
    mesh=_mesh,
    scratch_types=[
        pltpu.VMEM((NCH16, CH), jnp.int32),
        pltpu.VMEM((NCH16, CH), jnp.int32),
        pltpu.VMEM((CH, HD), _f32),
        pltpu.VMEM((CH, HD), _f32),
        pltpu.VMEM((CH, HD), _f32),
        pltpu.VMEM((CH, HD), _f32),
        pltpu.VMEM((128, HD), _f32),
        pltpu.VMEM_SHARED((NPAD, HD), _f32),
        [pltpu.SemaphoreType.DMA] * 4,
        [pltpu.SemaphoreType.DMA] * 4,
    ],
    compiler_params=pltpu.CompilerParams(use_tc_tiling_on_sc=False),
)
def _edge_kernel(yl_hbm, yr_hbm, src_hbm, dst_hbm, out0_hbm, out1_hbm,
                 src_v, dst_v, rows0, rows1, rows2, rows3, zrows_v, z_s,
                 gsems, ssems):
    c = lax.axis_index("c")
    s = lax.axis_index("s")
    pltpu.sync_copy(src_hbm.at[s], src_v)
    pltpu.sync_copy(dst_hbm.at[s], dst_v)

    def zfill(i, carry):
        zrows_v[i // 4, pl.ds((i % 4) * 16, 16)] = jnp.zeros((16,), _f32)
        return carry
    lax.fori_loop(0, 128 * 4, zfill, 0)

    def zcopy(k, carry):
        pltpu.sync_copy(zrows_v, z_s.at[pl.ds(s * PT + k * 128, 128)])
        return carry
    lax.fori_loop(0, PT // 128, zcopy, 0)
    plsc.subcore_barrier()

    def run(y_hbm):
        rows = [rows0, rows1, rows2, rows3]

        def g_wait(j, b):
            pltpu.make_async_copy(y_hbm.at[src_v.at[j]], rows[b],
                                  gsems[b]).wait()

        def g_fire(j, b):
            pltpu.async_copy(y_hbm.at[src_v.at[j]], rows[b], gsems[b])

        def s_fire(j, b):
            pltpu.async_copy(rows[b], z_s.at[dst_v.at[j]], ssems[b], add=True)

        def s_wait(j, b):
            pltpu.make_async_copy(rows[b], z_s.at[dst_v.at[j]],
                                  ssems[b]).wait()

        g_fire(0, 0)
        g_fire(1, 1)
        for b in range(4):
            if b >= 2:
                g_wait(b, b)
                s_fire(b, b)
                s_wait(b - 2, b - 2)
            else:
                g_wait(b, b)
                s_fire(b, b)
            if b + 2 < 4:
                g_fire(b + 2, b + 2)
            else:
                g_fire(b + 2, (b + 2) % 4)

        def body(j4, carry):
            for b in range(4):
                j = j4 * 4 + b
                g_wait(j, b)
                s_fire(j, b)
                b2 = (b + 2) % 4
                s_wait(j - 2, b2)
                g_fire(j + 2, b2)
            return carry
        lax.fori_loop(1, NCH16 // 4 - 1, body, 0)
        for b in range(4):
            j = NCH16 - 4 + b
            g_wait(j, b)
            s_fire(j, b)
            if j + 2 < NCH16:
                b2 = (b + 2) % 4
                s_wait(j - 2, b2)
                g_fire(j + 2, b2)
        for b in range(4):
            s_wait(NCH16 - 4 + b, b)

    @pl.when(c == 0)
    def _():
        run(yl_hbm)

    @pl.when(c == 1)
    def _():
        run(yr_hbm)

    plsc.subcore_barrier()

    @pl.when(c == 0)
    def _():
        pltpu.sync_copy(z_s.at[pl.ds(s * PT, PT)],
                        out0_hbm.at[pl.ds(s * PT, PT)])

    @pl.when(c == 1)
    def _():
        pltpu.sync_copy(z_s.at[pl.ds(s * PT, PT)],
                        out1_hbm.at[pl.ds(s * PT, PT)])


@functools.partial(
    pl.kernel,
    out_type=[jax.ShapeDtypeStruct((NPAD,), _f32),
              jax.ShapeDtypeStruct((NPAD,), _f32)],
    mesh=_mesh,
    scratch_types=[
        pltpu.VMEM((NCH32, CH), jnp.int32),
        pltpu.VMEM((NCH32, CH), jnp.int32),
        pltpu.VMEM((CH,), _f32),
        pltpu.VMEM((CH,), _f32),
        pltpu.VMEM((CH,), _f32),
        pltpu.VMEM((CH,), _f32),
        pltpu.VMEM((PT,), _f32),
        pltpu.VMEM_SHARED((NPAD,), _f32),
        [pltpu.SemaphoreType.DMA] * 4,
        [pltpu.SemaphoreType.DMA] * 4,
    ],
)
def _scalar_edge_kernel(t_hbm, src_hbm, dst_hbm, out0_hbm, out1_hbm,
                        src_v, dst_v, vals0, vals1, vals2, vals3, zbuf_v,
                        acc_s, gsems, ssems):
    c = lax.axis_index("c")
    s = lax.axis_index("s")
    wid = c * NS + s
    pltpu.sync_copy(src_hbm.at[wid], src_v)
    pltpu.sync_copy(dst_hbm.at[wid], dst_v)

    def zfill(i, carry):
        zbuf_v[pl.ds(i * 16, 16)] = jnp.zeros((16,), _f32)
        return carry
    lax.fori_loop(0, PT // 16, zfill, 0)

    pltpu.sync_copy(zbuf_v, acc_s.at[pl.ds(s * PT, PT)])
    plsc.subcore_barrier()

    vals = [vals0, vals1, vals2, vals3]

    def g_wait(j, b):
        pltpu.make_async_copy(t_hbm.at[src_v.at[j]], vals[b], gsems[b]).wait()

    def g_fire(j, b):
        pltpu.async_copy(t_hbm.at[src_v.at[j]], vals[b], gsems[b])

    def s_fire(j, b):
        pltpu.async_copy(vals[b], acc_s.at[dst_v.at[j]], ssems[b], add=True)

    def s_wait(j, b):
        pltpu.make_async_copy(vals[b], acc_s.at[dst_v.at[j]], ssems[b]).wait()

    g_fire(0, 0)
    g_fire(1, 1)
    for b in range(4):
        g_wait(b, b)
        s_fire(b, b)
        if b >= 2:
            s_wait(b - 2, b - 2)
        g_fire(b + 2, (b + 2) % 4)

    def body(j4, carry):
        for b in range(4):
            j = j4 * 4 + b
            g_wait(j, b)
            s_fire(j, b)
            b2 = (b + 2) % 4
            s_wait(j - 2, b2)
            g_fire(j + 2, b2)
        return carry
    lax.fori_loop(1, NCH32 // 4 - 1, body, 0)
    for b in range(4):
        j = NCH32 - 4 + b
        g_wait(j, b)
        s_fire(j, b)
        if j + 2 < NCH32:
            b2 = (b + 2) % 4
            s_wait(j - 2, b2)
            g_fire(j + 2, b2)
    for b in range(4):
        s_wait(NCH32 - 4 + b, b)

    plsc.subcore_barrier()

    @pl.when(c == 0)
    def _():
        pltpu.sync_copy(acc_s.at[pl.ds(s * PT, PT)],
                        out0_hbm.at[pl.ds(s * PT, PT)])

    @pl.when(c == 1)
    def _():
        pltpu.sync_copy(acc_s.at[pl.ds(s * PT, PT)],
                        out1_hbm.at[pl.ds(s * PT, PT)])


def _mm_scale_body(x_ref, w_ref, d0_ref, d1_ref, y_ref, dinv_ref):
    deg = d0_ref[...] + d1_ref[...] + 1.0
    dinv = lax.rsqrt(deg)
    dinv_ref[...] = dinv
    y_ref[...] = dinv * jnp.dot(x_ref[...], w_ref[...],
                                preferred_element_type=_f32,
                                precision=lax.Precision.HIGHEST)


def _first_layer(x, W1, d0, d1):
    return pl.pallas_call(
        _mm_scale_body,
        grid=(NBLK,),
        in_specs=[
            pl.BlockSpec((BM, D), lambda i: (i, 0)),
            pl.BlockSpec((D, D), lambda i: (0, 0)),
            pl.BlockSpec((BM, 1), lambda i: (i, 0)),
            pl.BlockSpec((BM, 1), lambda i: (i, 0)),
        ],
        out_specs=[
            pl.BlockSpec((BM, D), lambda i: (i, 0)),
            pl.BlockSpec((BM, 1), lambda i: (i, 0)),
        ],
        out_shape=[
            jax.ShapeDtypeStruct((N, D), _f32),
            jax.ShapeDtypeStruct((N, 1), _f32),
        ],
    )(x, W1, d0, d1)


def _layer_body(z0_ref, z1_ref, y_ref, dinv_ref, b_ref, w_ref, out_ref):
    dinv = dinv_ref[...]
    z = jnp.concatenate([z0_ref[...], z1_ref[...]], axis=1)
    h = dinv * (z + y_ref[...]) + b_ref[...]
    h = jnp.maximum(h, 0.0)
    out_ref[...] = dinv * jnp.dot(h, w_ref[...], preferred_element_type=_f32,
                                precision=lax.Precision.HIGHEST)


def _mid_layer(z0, z1, y, dinv, b, Wn):
    return pl.pallas_call(
        _layer_body,
        grid=(NBLK,),
        in_specs=[
            pl.BlockSpec((BM, HD), lambda i: (i, 0)),
            pl.BlockSpec((BM, HD), lambda i: (i, 0)),
            pl.BlockSpec((BM, D), lambda i: (i, 0)),
            pl.BlockSpec((BM, 1), lambda i: (i, 0)),
            pl.BlockSpec((1, D), lambda i: (0, 0)),
            pl.BlockSpec((D, D), lambda i: (0, 0)),
        ],
        out_specs=pl.BlockSpec((BM, D), lambda i: (i, 0)),
        out_shape=jax.ShapeDtypeStruct((N, D), _f32),
    )(z0, z1, y, dinv, b, Wn)


def _final_node_body(z0_ref, z1_ref, y_ref, dinv_ref, b_ref, w4_ref, lw_ref,
                     t_ref):
    dinv = dinv_ref[...]
    z = jnp.concatenate([z0_ref[...], z1_ref[...]], axis=1)
    h = dinv * (z + y_ref[...]) + b_ref[...]
    h = jnp.maximum(h, 0.0)
    v = jnp.dot(w4_ref[...], lw_ref[...], preferred_element_type=_f32,
                                precision=lax.Precision.HIGHEST)
    t_ref[...] = dinv * jnp.dot(h, v, preferred_element_type=_f32,
                                precision=lax.Precision.HIGHEST)


def _final_node(z0, z1, y, dinv, b, W4, lin_w):
    return pl.pallas_call(
        _final_node_body,
        grid=(NBLK,),
        in_specs=[
            pl.BlockSpec((BM, HD), lambda i: (i, 0)),
            pl.BlockSpec((BM, HD), lambda i: (i, 0)),
            pl.BlockSpec((BM, D), lambda i: (i, 0)),
            pl.BlockSpec((BM, 1), lambda i: (i, 0)),
            pl.BlockSpec((1, D), lambda i: (0, 0)),
            pl.BlockSpec((D, D), lambda i: (0, 0)),
            pl.BlockSpec((D, 1), lambda i: (0, 0)),
        ],
        out_specs=pl.BlockSpec((BM, 1), lambda i: (i, 0)),
        out_shape=jax.ShapeDtypeStruct((N, 1), _f32),
    )(z0, z1, y, dinv, b, W4, lin_w)


def _pool_body(u0_ref, u1_ref, t_ref, dinv_ref, batch_ref, b4_ref, lw_ref,
               lb_ref, out_ref, acc_ref, cnt_ref):
    i = pl.program_id(0)

    @pl.when(i == 0)
    def _():
        acc_ref[...] = jnp.zeros_like(acc_ref)
        cnt_ref[...] = jnp.zeros_like(cnt_ref)

    u = dinv_ref[...] * (u0_ref[...] + u1_ref[...] + t_ref[...])
    gids = lax.broadcasted_iota(jnp.int32, (BM, G), 1)
    mask = (batch_ref[...] == gids).astype(_f32)
    acc_ref[...] += jnp.sum(mask * u, axis=0, keepdims=True)
    cnt_ref[...] += jnp.sum(mask, axis=0, keepdims=True)

    @pl.when(i == pl.num_programs(0) - 1)
    def _():
        c = jnp.sum(b4_ref[...] * lw_ref[...]) + lb_ref[0, 0]
        res = acc_ref[...] / jnp.maximum(cnt_ref[...], 1.0) + c
        out_ref[...] = jnp.broadcast_to(res, (8, G))


def _pool(u0, u1, t, dinv, batch2d, b4r, lin_w, lb):
    return pl.pallas_call(
        _pool_body,
        grid=(NBLK,),
        in_specs=[
            pl.BlockSpec((BM, 1), lambda i: (i, 0)),
            pl.BlockSpec((BM, 1), lambda i: (i, 0)),
            pl.BlockSpec((BM, 1), lambda i: (i, 0)),
            pl.BlockSpec((BM, 1), lambda i: (i, 0)),
            pl.BlockSpec((BM, 1), lambda i: (i, 0)),
            pl.BlockSpec((D, 1), lambda i: (0, 0)),
            pl.BlockSpec((D, 1), lambda i: (0, 0)),
            pl.BlockSpec((1, 1), lambda i: (0, 0), memory_space=pltpu.SMEM),
        ],
        out_specs=pl.BlockSpec((8, G), lambda i: (0, 0)),
        out_shape=jax.ShapeDtypeStruct((8, G), _f32),
        scratch_shapes=[
            pltpu.VMEM((1, G), _f32),
            pltpu.VMEM((1, G), _f32),
        ],
    )(u0, u1, t, dinv, batch2d, b4r, lin_w, lb)


def kernel(x, edge_index, batch, W1, b1, W2, b2, W3, b3, W4, b4, lin_w, lin_b):
    src32 = edge_index[0].reshape(NW, NCH32, CH)
    dst32 = edge_index[1].reshape(NW, NCH32, CH)
    src16 = edge_index[0].reshape(NS, NCH16, CH)
    dst16 = edge_index[1].reshape(NS, NCH16, CH)

    deg0, deg1 = _deg_kernel(dst32)
    d0 = deg0[:N, None]
    d1 = deg1[:N, None]

    y1, dinv = _first_layer(x, W1, d0, d1)
    z0, z1 = _edge_kernel(y1[:, :HD], y1[:, HD:], src16, dst16)
    y2 = _mid_layer(z0[:N], z1[:N], y1, dinv, b1.reshape(1, D), W2)
    z0, z1 = _edge_kernel(y2[:, :HD], y2[:, HD:], src16, dst16)
    y3 = _mid_layer(z0[:N], z1[:N], y2, dinv, b2.reshape(1, D), W3)
    z0, z1 = _edge_kernel(y3[:, :HD], y3[:, HD:], src16, dst16)
    t = _final_node(z0[:N], z1[:N], y3, dinv, b3.reshape(1, D), W4, lin_w)

    u0, u1 = _scalar_edge_kernel(t.reshape(N), src32, dst32)
    out8 = _pool(u0[:N, None], u1[:N, None], t, dinv,
                 batch.reshape(N, 1), b4.reshape(D, 1), lin_w,
                 lin_b.reshape(1, 1))
    return out8[0].reshape(G, 1)


# speedup vs baseline: 25.3449x; 1.0485x over previous
import functools

import jax
import jax.numpy as jnp
from jax import lax
from jax.experimental import pallas as pl
from jax.experimental.pallas import tpu as pltpu
from jax.experimental.pallas import tpu_sc as plsc

N = 10000
D = 128
HD = D // 2
E = 320000
G = 64
NC = 2
NS = 16
NW = NC * NS
CH = 125
NCH32 = E // NW // CH
NCH16 = E // NS // CH
NPAD = 10240
PT = NPAD // NS
NBUF = 5
LK = 2

BM = 2000
NBLK = N // BM

_mesh = plsc.VectorSubcoreMesh(
    core_axis_name="c", subcore_axis_name="s", num_cores=NC, num_subcores=NS)

_f32 = jnp.float32


def _ring(nch, g_fire, g_wait, s_fire, s_wait):
    dr = NBUF - LK
    for j in range(LK):
        g_fire(j, j)
    for j in range(NBUF):
        g_wait(j, j)
        s_fire(j, j)
        if j >= dr:
            s_wait(j - dr, j - dr)
        g_fire(j + LK, (j + LK) % NBUF)

    def body(jg, carry):
        for b in range(NBUF):
            j = jg * NBUF + b
            g_wait(j, b)
            s_fire(j, b)
            bb = (b + LK) % NBUF
            s_wait(j + LK - NBUF, bb)
            g_fire(j + LK, bb)
        return carry
    lax.fori_loop(1, nch // NBUF - 1, body, 0)
    for b in range(NBUF):
        j = nch - NBUF + b
        g_wait(j, b)
        s_fire(j, b)
        if j + LK < nch:
            bb = (b + LK) % NBUF
            s_wait(j + LK - NBUF, bb)
            g_fire(j + LK, bb)
    for b in range(NBUF):
        s_wait(nch - NBUF + b, b)


@functools.partial(
    pl.kernel,
    out_type=[jax.ShapeDtypeStruct((NPAD,), _f32),
              jax.ShapeDtypeStruct((NPAD,), _f32)],
    mesh=_mesh,
    scratch_types=[
        pltpu.VMEM((NCH32, CH), jnp.int32),
        pltpu.VMEM((PT,), _f32),
        pltpu.VMEM((128,), _f32),
        pltpu.VMEM_SHARED((NPAD,), _f32),
    ],
)
def _deg_kernel(dst_hbm, out0_hbm, out1_hbm, dst_v, zbuf_v, ones_v, acc_s):
    c = lax.axis_index("c")
    s = lax.axis_index("s")
    wid = c * NS + s
    pltpu.sync_copy(dst_hbm.at[wid], dst_v)

    def zfill(i, carry):
        zbuf_v[pl.ds(i * 16, 16)] = jnp.zeros((16,), _f32)
        return carry
    lax.fori_loop(0, PT // 16, zfill, 0)

    def ofill(i, carry):
        ones_v[pl.ds(i * 16, 16)] = jnp.ones((16,), _f32)
        return carry
    lax.fori_loop(0, 8, ofill, 0)

    pltpu.sync_copy(zbuf_v, acc_s.at[pl.ds(s * PT, PT)])
    plsc.subcore_barrier()

    def body(j, carry):
        pltpu.sync_copy(ones_v.at[pl.ds(0, CH)], acc_s.at[dst_v.at[j]],
                        add=True)
        return carry
    lax.fori_loop(0, NCH32, body, 0)

    plsc.subcore_barrier()

    @pl.when(c == 0)
    def _():
        pltpu.sync_copy(acc_s.at[pl.ds(s * PT, PT)],
                        out0_hbm.at[pl.ds(s * PT, PT)])

    @pl.when(c == 1)
    def _():
        pltpu.sync_copy(acc_s.at[pl.ds(s * PT, PT)],
                        out1_hbm.at[pl.ds(s * PT, PT)])


@functools.partial(
    pl.kernel,
    out_type=[jax.ShapeDtypeStruct((NPAD, HD), _f32),
              jax.ShapeDtypeStruct((NPAD, HD), _f32)],
    mesh=_mesh,
    scratch_types=[
        pltpu.VMEM((NCH16, CH), jnp.int32),
        pltpu.VMEM((NCH16, CH), jnp.int32),
        [pltpu.VMEM((CH, HD), _f32)] * NBUF,
        pltpu.VMEM((128, HD), _f32),
        pltpu.VMEM_SHARED((NPAD, HD), _f32),
        [pltpu.SemaphoreType.DMA] * NBUF,
        [pltpu.SemaphoreType.DMA] * NBUF,
    ],
    compiler_params=pltpu.CompilerParams(use_tc_tiling_on_sc=False),
)
def _edge_kernel(yl_hbm, yr_hbm, src_hbm, dst_hbm, out0_hbm, out1_hbm,
                 src_v, dst_v, rows, zrows_v, z_s, gsems, ssems):
    c = lax.axis_index("c")
    s = lax.axis_index("s")
    pltpu.sync_copy(src_hbm.at[s], src_v)
    pltpu.sync_copy(dst_hbm.at[s], dst_v)

    def zfill(i, carry):
        zrows_v[i // 4, pl.ds((i % 4) * 16, 16)] = jnp.zeros((16,), _f32)
        return carry
    lax.fori_loop(0, 128 * 4, zfill, 0)

    def zcopy(k, carry):
        pltpu.sync_copy(zrows_v, z_s.at[pl.ds(s * PT + k * 128, 128)])
        return carry
    lax.fori_loop(0, PT // 128, zcopy, 0)
    plsc.subcore_barrier()

    def run(y_hbm):
        def g_wait(j, b):
            pltpu.make_async_copy(y_hbm.at[src_v.at[j]], rows[b],
                                  gsems[b]).wait()

        def g_fire(j, b):
            pltpu.async_copy(y_hbm.at[src_v.at[j]], rows[b], gsems[b])

        def s_fire(j, b):
            pltpu.async_copy(rows[b], z_s.at[dst_v.at[j]], ssems[b], add=True)

        def s_wait(j, b):
            pltpu.make_async_copy(rows[b], z_s.at[dst_v.at[j]],
                                  ssems[b]).wait()

        _ring(NCH16, g_fire, g_wait, s_fire, s_wait)

    @pl.when(c == 0)
    def _():
        run(yl_hbm)

    @pl.when(c == 1)
    def _():
        run(yr_hbm)

    plsc.subcore_barrier()

    @pl.when(c == 0)
    def _():
        pltpu.sync_copy(z_s.at[pl.ds(s * PT, PT)],
                        out0_hbm.at[pl.ds(s * PT, PT)])

    @pl.when(c == 1)
    def _():
        pltpu.sync_copy(z_s.at[pl.ds(s * PT, PT)],
                        out1_hbm.at[pl.ds(s * PT, PT)])


@functools.partial(
    pl.kernel,
    out_type=[jax.ShapeDtypeStruct((NPAD,), _f32),
              jax.ShapeDtypeStruct((NPAD,), _f32)],
    mesh=_mesh,
    scratch_types=[
        pltpu.VMEM((NCH32, CH), jnp.int32),
        pltpu.VMEM((NCH32, CH), jnp.int32),
        [pltpu.VMEM((CH,), _f32)] * NBUF,
        pltpu.VMEM((PT,), _f32),
        pltpu.VMEM_SHARED((NPAD,), _f32),
        [pltpu.SemaphoreType.DMA] * NBUF,
        [pltpu.SemaphoreType.DMA] * NBUF,
    ],
)
def _scalar_edge_kernel(t_hbm, src_hbm, dst_hbm, out0_hbm, out1_hbm,
                        src_v, dst_v, vals, zbuf_v, acc_s, gsems, ssems):
    c = lax.axis_index("c")
    s = lax.axis_index("s")
    wid = c * NS + s
    pltpu.sync_copy(src_hbm.at[wid], src_v)
    pltpu.sync_copy(dst_hbm.at[wid], dst_v)

    def zfill(i, carry):
        zbuf_v[pl.ds(i * 16, 16)] = jnp.zeros((16,), _f32)
        return carry
    lax.fori_loop(0, PT // 16, zfill, 0)

    pltpu.sync_copy(zbuf_v, acc_s.at[pl.ds(s * PT, PT)])
    plsc.subcore_barrier()

    def g_wait(j, b):
        pltpu.make_async_copy(t_hbm.at[src_v.at[j]], vals[b], gsems[b]).wait()

    def g_fire(j, b):
        pltpu.async_copy(t_hbm.at[src_v.at[j]], vals[b], gsems[b])

    def s_fire(j, b):
        pltpu.async_copy(vals[b], acc_s.at[dst_v.at[j]], ssems[b], add=True)

    def s_wait(j, b):
        pltpu.make_async_copy(vals[b], acc_s.at[dst_v.at[j]], ssems[b]).wait()

    _ring(NCH32, g_fire, g_wait, s_fire, s_wait)

    plsc.subcore_barrier()

    @pl.when(c == 0)
    def _():
        pltpu.sync_copy(acc_s.at[pl.ds(s * PT, PT)],
                        out0_hbm.at[pl.ds(s * PT, PT)])

    @pl.when(c == 1)
    def _():
        pltpu.sync_copy(acc_s.at[pl.ds(s * PT, PT)],
                        out1_hbm.at[pl.ds(s * PT, PT)])


_ROW = pl.BlockSpec((BM, 1), lambda i: (i, 0))


def _mm_scale_body(x_ref, w_ref, d0_ref, d1_ref, yl_ref, yr_ref, dinv_ref):
    deg = d0_ref[...] + d1_ref[...] + 1.0
    dinv = lax.rsqrt(deg)
    dinv_ref[...] = dinv
    y = dinv * jnp.dot(x_ref[...], w_ref[...], preferred_element_type=_f32)
    yl_ref[...] = y[:, :HD]
    yr_ref[...] = y[:, HD:]


def _first_layer(x, W1, d0, d1):
    return pl.pallas_call(
        _mm_scale_body,
        grid=(NBLK,),
        in_specs=[
            pl.BlockSpec((BM, D), lambda i: (i, 0)),
            pl.BlockSpec((D, D), lambda i: (0, 0)),
            _ROW,
            _ROW,
        ],
        out_specs=[
            pl.BlockSpec((BM, HD), lambda i: (i, 0)),
            pl.BlockSpec((BM, HD), lambda i: (i, 0)),
            _ROW,
        ],
        out_shape=[
            jax.ShapeDtypeStruct((N, HD), _f32),
            jax.ShapeDtypeStruct((N, HD), _f32),
            jax.ShapeDtypeStruct((N, 1), _f32),
        ],
    )(x, W1, d0, d1)


def _layer_body(z0_ref, z1_ref, yl_ref, yr_ref, dinv_ref, b_ref, w_ref,
                ol_ref, or_ref):
    dinv = dinv_ref[...]
    zl = z0_ref[...] + yl_ref[...]
    zr = z1_ref[...] + yr_ref[...]
    h = dinv * jnp.concatenate([zl, zr], axis=1) + b_ref[...]
    h = jnp.maximum(h, 0.0)
    y = dinv * jnp.dot(h, w_ref[...], preferred_element_type=_f32)
    ol_ref[...] = y[:, :HD]
    or_ref[...] = y[:, HD:]


def _mid_layer(z0, z1, yl, yr, dinv, b, Wn):
    return pl.pallas_call(
        _layer_body,
        grid=(NBLK,),
        in_specs=[
            pl.BlockSpec((BM, HD), lambda i: (i, 0)),
            pl.BlockSpec((BM, HD), lambda i: (i, 0)),
            pl.BlockSpec((BM, HD), lambda i: (i, 0)),
            pl.BlockSpec((BM, HD), lambda i: (i, 0)),
            _ROW,
            pl.BlockSpec((1, D), lambda i: (0, 0)),
            pl.BlockSpec((D, D), lambda i: (0, 0)),
        ],
        out_specs=[
            pl.BlockSpec((BM, HD), lambda i: (i, 0)),
            pl.BlockSpec((BM, HD), lambda i: (i, 0)),
        ],
        out_shape=[
            jax.ShapeDtypeStruct((N, HD), _f32),
            jax.ShapeDtypeStruct((N, HD), _f32),
        ],
    )(z0, z1, yl, yr, dinv, b, Wn)


def _final_node_body(z0_ref, z1_ref, yl_ref, yr_ref, dinv_ref, b_ref, w4_ref,
                     lw_ref, t_ref):
    dinv = dinv_ref[...]
    zl = z0_ref[...] + yl_ref[...]
    zr = z1_ref[...] + yr_ref[...]
    h = dinv * jnp.concatenate([zl, zr], axis=1) + b_ref[...]
    h = jnp.maximum(h, 0.0)
    v = jnp.dot(w4_ref[...], lw_ref[...], preferred_element_type=_f32)
    t_ref[...] = dinv * jnp.dot(h, v, preferred_element_type=_f32)


def _final_node(z0, z1, yl, yr, dinv, b, W4, lin_w):
    return pl.pallas_call(
        _final_node_body,
        grid=(NBLK,),
        in_specs=[
            pl.BlockSpec((BM, HD), lambda i: (i, 0)),
            pl.BlockSpec((BM, HD), lambda i: (i, 0)),
            pl.BlockSpec((BM, HD), lambda i: (i, 0)),
            pl.BlockSpec((BM, HD), lambda i: (i, 0)),
            _ROW,
            pl.BlockSpec((1, D), lambda i: (0, 0)),
            pl.BlockSpec((D, D), lambda i: (0, 0)),
            pl.BlockSpec((D, 1), lambda i: (0, 0)),
        ],
        out_specs=_ROW,
        out_shape=jax.ShapeDtypeStruct((N, 1), _f32),
    )(z0, z1, yl, yr, dinv, b, W4, lin_w)


def _pool_body(u0_ref, u1_ref, t_ref, dinv_ref, batch_ref, b4_ref, lw_ref,
               lb_ref, out_ref, acc_ref, cnt_ref):
    i = pl.program_id(0)

    @pl.when(i == 0)
    def _():
        acc_ref[...] = jnp.zeros_like(acc_ref)
        cnt_ref[...] = jnp.zeros_like(cnt_ref)

    u = dinv_ref[...] * (u0_ref[...] + u1_ref[...] + t_ref[...])
    gids = lax.broadcasted_iota(jnp.int32, (BM, G), 1)
    mask = (batch_ref[...] == gids).astype(_f32)
    acc_ref[...] += jnp.sum(mask * u, axis=0, keepdims=True)
    cnt_ref[...] += jnp.sum(mask, axis=0, keepdims=True)

    @pl.when(i == pl.num_programs(0) - 1)
    def _():
        c = jnp.sum(b4_ref[...] * lw_ref[...]) + lb_ref[0, 0]
        res = acc_ref[...] / jnp.maximum(cnt_ref[...], 1.0) + c
        out_ref[...] = jnp.broadcast_to(res, (8, G))


def _pool(u0, u1, t, dinv, batch2d, b4r, lin_w, lb):
    return pl.pallas_call(
        _pool_body,
        grid=(NBLK,),
        in_specs=[
            _ROW,
            _ROW,
            _ROW,
            _ROW,
            _ROW,
            pl.BlockSpec((D, 1), lambda i: (0, 0)),
            pl.BlockSpec((D, 1), lambda i: (0, 0)),
            pl.BlockSpec((1, 1), lambda i: (0, 0), memory_space=pltpu.SMEM),
        ],
        out_specs=pl.BlockSpec((8, G), lambda i: (0, 0)),
        out_shape=jax.ShapeDtypeStruct((8, G), _f32),
        scratch_shapes=[
            pltpu.VMEM((1, G), _f32),
            pltpu.VMEM((1, G), _f32),
        ],
    )(u0, u1, t, dinv, batch2d, b4r, lin_w, lb)


def kernel(x, edge_index, batch, W1, b1, W2, b2, W3, b3, W4, b4, lin_w, lin_b):
    src32 = edge_index[0].reshape(NW, NCH32, CH)
    dst32 = edge_index[1].reshape(NW, NCH32, CH)
    src16 = edge_index[0].reshape(NS, NCH16, CH)
    dst16 = edge_index[1].reshape(NS, NCH16, CH)

    deg0, deg1 = _deg_kernel(dst32)
    d0 = deg0.reshape(NPAD, 1)
    d1 = deg1.reshape(NPAD, 1)

    yl, yr, dinv = _first_layer(x, W1, d0, d1)
    z0, z1 = _edge_kernel(yl, yr, src16, dst16)
    yl, yr = _mid_layer(z0, z1, yl, yr, dinv, b1.reshape(1, D), W2)
    z0, z1 = _edge_kernel(yl, yr, src16, dst16)
    yl, yr = _mid_layer(z0, z1, yl, yr, dinv, b2.reshape(1, D), W3)
    z0, z1 = _edge_kernel(yl, yr, src16, dst16)
    t = _final_node(z0, z1, yl, yr, dinv, b3.reshape(1, D), W4, lin_w)

    u0, u1 = _scalar_edge_kernel(t.reshape(N), src32, dst32)
    out8 = _pool(u0.reshape(NPAD, 1), u1.reshape(NPAD, 1), t, dinv,
                 batch.reshape(N, 1), b4.reshape(D, 1), lin_w,
                 lin_b.reshape(1, 1))
    return out8[0].reshape(G, 1)
